# Initial kernel scaffold; baseline (speedup 1.0000x reference)
import jax
import jax.numpy as jnp
from jax.experimental import pallas as pl


def kernel(inputs, coords, edge_index, params):
    raise NotImplementedError("write your pallas kernel here")


# speedup vs baseline: 5.0457x; 5.0457x over previous
import dataclasses
import functools

import jax
import jax.numpy as jnp
from jax.experimental import pallas as pl
from jax.experimental.pallas import tpu as pltpu
from jax.experimental.pallas import tpu_sc as plsc

N = 10000
E = 160000
E2 = 2 * E
WID = 128
GW = 128
SW = 128
ER = 2000

_F32 = jnp.float32


def _vmesh():
    return plsc.VectorSubcoreMesh(core_axis_name="c", subcore_axis_name="s")


def _sc_params():
    cp = pltpu.CompilerParams()
    if "needs_layout_passes" in pltpu.CompilerParams.__dataclass_fields__:
        cp = dataclasses.replace(cp, needs_layout_passes=False)
    return cp


def _sc_gather_pair(tsrc, tdst, isrc, idst):
    tw = tsrc.shape[1]
    n_idx = isrc.shape[1]

    @pl.kernel(
        out_type=[
            jax.ShapeDtypeStruct((n_idx, tw), _F32),
            jax.ShapeDtypeStruct((n_idx, tw), _F32),
        ],
        mesh=_vmesh(),
    )
    def gk(tsrc_hbm, tdst_hbm, isrc_hbm, idst_hbm, osrc_hbm, odst_hbm):
        def body(isrc_v, idst_v, osrc_v, odst_v):
            pltpu.sync_copy(tsrc_hbm.at[isrc_v.at[0]], osrc_v)
            pltpu.sync_copy(tdst_hbm.at[idst_v.at[0]], odst_v)

        pltpu.emit_pipeline(
            body,
            grid=(n_idx // GW,),
            in_specs=[
                pl.BlockSpec((1, GW), lambda i: (0, i)),
                pl.BlockSpec((1, GW), lambda i: (0, i)),
            ],
            out_specs=[
                pl.BlockSpec((GW, tw), lambda i: (i, 0)),
                pl.BlockSpec((GW, tw), lambda i: (i, 0)),
            ],
            core_axis_name=("c", "s"),
            dimension_semantics=(pltpu.PARALLEL,),
        )(isrc_hbm, idst_hbm, osrc_hbm, odst_hbm)

    return gk(tsrc, tdst, isrc, idst)


def _sc_gather1(tsrc, tdst, cx, cy, cz, isrc, idst):
    tw = tsrc.shape[1]
    n_idx = isrc.shape[1]
    geo = jax.ShapeDtypeStruct((n_idx,), _F32)

    @pl.kernel(
        out_type=[
            jax.ShapeDtypeStruct((n_idx, tw), _F32),
            jax.ShapeDtypeStruct((n_idx, tw), _F32),
            geo, geo, geo, geo,
        ],
        mesh=_vmesh(),
        scratch_types=[
            pltpu.VMEM((N,), _F32),
            pltpu.VMEM((N,), _F32),
            pltpu.VMEM((N,), _F32),
        ],
        compiler_params=_sc_params(),
    )
    def gk(tsrc_hbm, tdst_hbm, cx_hbm, cy_hbm, cz_hbm, isrc_hbm, idst_hbm,
           osrc_hbm, odst_hbm, orx_hbm, ory_hbm, orz_hbm, od2_hbm,
           cx_v, cy_v, cz_v):
        pltpu.sync_copy(cx_hbm, cx_v)
        pltpu.sync_copy(cy_hbm, cy_v)
        pltpu.sync_copy(cz_hbm, cz_v)

        def body(isrc_v, idst_v, osrc_v, odst_v, orx_v, ory_v, orz_v, od2_v):
            pltpu.sync_copy(tsrc_hbm.at[isrc_v.at[0]], osrc_v)
            pltpu.sync_copy(tdst_hbm.at[idst_v.at[0]], odst_v)

            @pl.loop(0, GW, step=16)
            def _(c):
                s16 = isrc_v[0, pl.ds(c, 16)]
                d16 = idst_v[0, pl.ds(c, 16)]
                rx = (plsc.load_gather(cx_v, [d16])
                      - plsc.load_gather(cx_v, [s16]))
                ry = (plsc.load_gather(cy_v, [d16])
                      - plsc.load_gather(cy_v, [s16]))
                rz = (plsc.load_gather(cz_v, [d16])
                      - plsc.load_gather(cz_v, [s16]))
                orx_v[pl.ds(c, 16)] = rx
                ory_v[pl.ds(c, 16)] = ry
                orz_v[pl.ds(c, 16)] = rz
                od2_v[pl.ds(c, 16)] = rx * rx + ry * ry + rz * rz

        pltpu.emit_pipeline(
            body,
            grid=(n_idx // GW,),
            in_specs=[
                pl.BlockSpec((1, GW), lambda i: (0, i)),
                pl.BlockSpec((1, GW), lambda i: (0, i)),
            ],
            out_specs=[
                pl.BlockSpec((GW, tw), lambda i: (i, 0)),
                pl.BlockSpec((GW, tw), lambda i: (i, 0)),
                pl.BlockSpec((GW,), lambda i: (i,)),
                pl.BlockSpec((GW,), lambda i: (i,)),
                pl.BlockSpec((GW,), lambda i: (i,)),
                pl.BlockSpec((GW,), lambda i: (i,)),
            ],
            core_axis_name=("c", "s"),
            dimension_semantics=(pltpu.PARALLEL,),
        )(isrc_hbm, idst_hbm, osrc_hbm, odst_hbm,
          orx_hbm, ory_hbm, orz_hbm, od2_hbm)

    return gk(tsrc, tdst, cx, cy, cz, isrc, idst)


def _sc_scatter_add(u, idx):
    n_idx = idx.shape[1]
    z_tab = jnp.zeros((N, WID), _F32)

    @pl.kernel(
        out_type=jax.ShapeDtypeStruct((2, N, WID), _F32),
        mesh=_vmesh(),
        scratch_types=[pltpu.VMEM_SHARED((N, WID), _F32)],
    )
    def sk(u_hbm, idx_hbm, z_hbm, s_hbm, tab):
        core = jax.lax.axis_index("c")
        sid = jax.lax.axis_index("s")

        @pl.when(sid == 0)
        def _():
            pltpu.sync_copy(z_hbm, tab)

        plsc.subcore_barrier()

        def body(u_v, idx_v):
            pltpu.sync_copy(u_v, tab.at[idx_v.at[0]], add=True)

        pltpu.emit_pipeline(
            body,
            grid=(n_idx // SW,),
            in_specs=[
                pl.BlockSpec((SW, WID), lambda i: (i, 0)),
                pl.BlockSpec((1, SW), lambda i: (0, i)),
            ],
            core_axis_name=("c", "s"),
            dimension_semantics=(pltpu.PARALLEL,),
        )(u_hbm, idx_hbm)

        plsc.subcore_barrier()

        @pl.when(sid == 0)
        def _():
            pltpu.sync_copy(tab, s_hbm.at[core])

    return sk(u, idx, z_tab)


def _sc_degree(idx):
    n_idx = idx.shape[1]
    z_deg = jnp.zeros((N, WID), _F32)
    ones = jnp.ones((SW, WID), _F32)

    @pl.kernel(
        out_type=jax.ShapeDtypeStruct((2, N, WID), _F32),
        mesh=_vmesh(),
        scratch_types=[pltpu.VMEM_SHARED((N, WID), _F32)],
    )
    def dk(ones_hbm, idx_hbm, zd_hbm, d_hbm, dtab):
        core = jax.lax.axis_index("c")
        sid = jax.lax.axis_index("s")

        @pl.when(sid == 0)
        def _():
            pltpu.sync_copy(zd_hbm, dtab)

        plsc.subcore_barrier()

        def body(ones_v, idx_v):
            pltpu.sync_copy(ones_v, dtab.at[idx_v.at[0]], add=True)

        pltpu.emit_pipeline(
            body,
            grid=(n_idx // SW,),
            in_specs=[
                pl.BlockSpec((SW, WID), lambda i: (0, 0)),
                pl.BlockSpec((1, SW), lambda i: (0, i)),
            ],
            core_axis_name=("c", "s"),
            dimension_semantics=(pltpu.PARALLEL,),
        )(ones_hbm, idx_hbm)

        plsc.subcore_barrier()

        @pl.when(sid == 0)
        def _():
            pltpu.sync_copy(dtab, d_hbm.at[core])

    return dk(ones, idx, z_deg)


def _gelu(x):
    return x * 0.5 * (1.0 + jax.lax.erf(x * 0.7071067811865476))


def _dot(a, b):
    return jax.lax.dot_general(a, b, (((1,), (0,)), ((), ())),
                               preferred_element_type=_F32)


def _ln(u, g, b):
    mu = jnp.mean(u, axis=-1, keepdims=True)
    var = jnp.mean((u - mu) * (u - mu), axis=-1, keepdims=True)
    return (u - mu) * jax.lax.rsqrt(var + 1e-5) * g + b


def _full_spec(shape):
    return pl.BlockSpec(shape, lambda *_: tuple(0 for _ in shape))


def _tc_node_encode(cin8, we1, be1, we2, be2, we3, be3, w1a, w1b):
    def body(cin_r, we1_r, be1_r, we2_r, be2_r, we3_r, be3_r,
             w1a_r, w1b_r, x0_r, ts_r, td_r):
        h = _gelu(_dot(cin_r[...], we1_r[...]) + be1_r[...])
        h = _gelu(_dot(h, we2_r[...]) + be2_r[...])
        x0 = _dot(h, we3_r[...]) + be3_r[...]
        x0_r[...] = x0
        ts_r[...] = _dot(x0, w1a_r[...])
        td_r[...] = _dot(x0, w1b_r[...])

    return pl.pallas_call(
        body,
        out_shape=[
            jax.ShapeDtypeStruct((N, WID), _F32),
            jax.ShapeDtypeStruct((N, WID), _F32),
            jax.ShapeDtypeStruct((N, WID), _F32),
        ],
    )(cin8, we1, be1, we2, be2, we3, be3, w1a, w1b)


def _tc_edge1(gsrc, gdst, relx, rely, relz, d2,
              wr0, wr1, wr2, wd, bee1, we2, be2, we3, be3,
              w1c, b1, w2, b2, w3, b3, g, bn):
    def body(gs_r, gd_r, rx_r, ry_r, rz_r, d2_r,
             wr0_r, wr1_r, wr2_r, wd_r, bee1_r, we2_r, be2_r, we3_r, be3_r,
             w1c_r, b1_r, w2_r, b2_r, w3_r, b3_r, g_r, bn_r, u_r):
        dist = jnp.sqrt(d2_r[...])
        h = _gelu(rx_r[...] * wr0_r[...] + ry_r[...] * wr1_r[...]
                  + rz_r[...] * wr2_r[...] + dist * wd_r[...] + bee1_r[...])
        h = _gelu(_dot(h, we2_r[...]) + be2_r[...])
        ea = _dot(h, we3_r[...]) + be3_r[...]
        z = _gelu(_dot(ea, w1c_r[...]) + gs_r[...] + gd_r[...] + b1_r[...])
        z = _gelu(_dot(z, w2_r[...]) + b2_r[...])
        u = _dot(z, w3_r[...]) + b3_r[...]
        u_r[...] = _ln(u, g_r[...], bn_r[...])

    col = pl.BlockSpec((ER, 1), lambda i: (i, 0))
    return pl.pallas_call(
        body,
        grid=(E2 // ER,),
        in_specs=[
            pl.BlockSpec((ER, WID), lambda i: (i, 0)),
            pl.BlockSpec((ER, WID), lambda i: (i, 0)),
            col, col, col, col,
            _full_spec((1, WID)), _full_spec((1, WID)), _full_spec((1, WID)),
            _full_spec((1, WID)), _full_spec((1, WID)),
            _full_spec((WID, WID)), _full_spec((1, WID)),
            _full_spec((WID, WID)), _full_spec((1, WID)),
            _full_spec((WID, WID)), _full_spec((1, WID)),
            _full_spec((WID, WID)), _full_spec((1, WID)),
            _full_spec((WID, WID)), _full_spec((1, WID)),
            _full_spec((1, WID)), _full_spec((1, WID)),
        ],
        out_specs=pl.BlockSpec((ER, WID), lambda i: (i, 0)),
        out_shape=jax.ShapeDtypeStruct((E2, WID), _F32),
    )(gsrc, gdst, relx, rely, relz, d2,
      wr0, wr1, wr2, wd, bee1, we2, be2, we3, be3,
      w1c, b1, w2, b2, w3, b3, g, bn)


def _tc_edge2(attr, gsrc, gdst, w1c, b1, w2, b2, w3, b3, g, bn):
    def body(a_r, gs_r, gd_r, w1c_r, b1_r, w2_r, b2_r, w3_r, b3_r,
             g_r, bn_r, u_r):
        z = _gelu(_dot(a_r[...], w1c_r[...]) + gs_r[...] + gd_r[...]
                  + b1_r[...])
        z = _gelu(_dot(z, w2_r[...]) + b2_r[...])
        u = _dot(z, w3_r[...]) + b3_r[...]
        u_r[...] = _ln(u, g_r[...], bn_r[...])

    return pl.pallas_call(
        body,
        grid=(E2 // ER,),
        in_specs=[
            pl.BlockSpec((ER, WID), lambda i: (i, 0)),
            pl.BlockSpec((ER, WID), lambda i: (i, 0)),
            pl.BlockSpec((ER, WID), lambda i: (i, 0)),
            _full_spec((WID, WID)), _full_spec((1, WID)),
            _full_spec((WID, WID)), _full_spec((1, WID)),
            _full_spec((WID, WID)), _full_spec((1, WID)),
            _full_spec((1, WID)), _full_spec((1, WID)),
        ],
        out_specs=pl.BlockSpec((ER, WID), lambda i: (i, 0)),
        out_shape=jax.ShapeDtypeStruct((E2, WID), _F32),
    )(attr, gsrc, gdst, w1c, b1, w2, b2, w3, b3, g, bn)


def _tc_node1(x0, s, dg, wna, wnb, bn1, wn2, bn2, wn3, bn3, g, b,
              w1a2, w1b2):
    def body(x_r, s_r, dg_r, wna_r, wnb_r, bn1_r, wn2_r, bn2_r, wn3_r,
             bn3_r, g_r, b_r, w1a_r, w1b_r, x1_r, p_r, q_r):
        x0v = x_r[...]
        deg = dg_r[0, :, 0:1] + dg_r[1, :, 0:1]
        agg = (s_r[0] + s_r[1]) / jnp.maximum(deg, 1.0)
        h = _gelu(_dot(x0v, wna_r[...]) + _dot(agg, wnb_r[...]) + bn1_r[...])
        h = _gelu(_dot(h, wn2_r[...]) + bn2_r[...])
        nu = _dot(h, wn3_r[...]) + bn3_r[...]
        x1 = x0v + _ln(nu, g_r[...], b_r[...])
        x1_r[...] = x1
        p_r[...] = _dot(x1, w1a_r[...])
        q_r[...] = _dot(x1, w1b_r[...])

    return pl.pallas_call(
        body,
        out_shape=[
            jax.ShapeDtypeStruct((N, WID), _F32),
            jax.ShapeDtypeStruct((N, WID), _F32),
            jax.ShapeDtypeStruct((N, WID), _F32),
        ],
    )(x0, s, dg, wna, wnb, bn1, wn2, bn2, wn3, bn3, g, b, w1a2, w1b2)


def _tc_node2(x1, s, dg, wna, wnb, bn1, wn2, bn2, wn3, bn3, g, b,
              wp1, bp1, wp2, bp2):
    def body(x_r, s_r, dg_r, wna_r, wnb_r, bn1_r, wn2_r, bn2_r, wn3_r,
             bn3_r, g_r, b_r, wp1_r, bp1_r, wp2_r, bp2_r, o_r):
        x1v = x_r[...]
        deg = dg_r[0, :, 0:1] + dg_r[1, :, 0:1]
        agg = (s_r[0] + s_r[1]) / jnp.maximum(deg, 1.0)
        h = _gelu(_dot(x1v, wna_r[...]) + _dot(agg, wnb_r[...]) + bn1_r[...])
        h = _gelu(_dot(h, wn2_r[...]) + bn2_r[...])
        nu = _dot(h, wn3_r[...]) + bn3_r[...]
        x2 = x1v + _ln(nu, g_r[...], b_r[...])
        o = _gelu(_dot(x2, wp1_r[...]) + bp1_r[...])
        o_r[...] = _dot(o, wp2_r[...]) + bp2_r[...]

    return pl.pallas_call(
        body,
        out_shape=jax.ShapeDtypeStruct((N, 4), _F32),
    )(x1, s, dg, wna, wnb, bn1, wn2, bn2, wn3, bn3, g, b, wp1, bp1, wp2, bp2)


def _row(v):
    return v.reshape(1, -1)


def kernel(inputs, coords, edge_index, params):
    c2 = coords[0]
    f2 = inputs[0]

    ei = edge_index.astype(jnp.int32)
    src = jnp.concatenate([ei[0], ei[1]])
    dst = jnp.concatenate([ei[1], ei[0]])
    isrc = src.reshape(1, E2)
    idst = dst.reshape(1, E2)

    cin8 = jnp.concatenate([c2, f2, jnp.zeros((N, 1), _F32)], axis=-1)

    enc = params["node_encoder"]
    we1 = jnp.concatenate([enc[0]["W"], jnp.zeros((1, WID), _F32)], axis=0)
    eenc = params["edge_encoder"]
    wr0 = _row(eenc[0]["W"][0])
    wr1 = _row(eenc[0]["W"][1])
    wr2 = _row(eenc[0]["W"][2])
    wd = _row(eenc[0]["W"][3])

    b1, b2 = params["blocks"]
    e1, e2m = b1["edge_mlp"], b2["edge_mlp"]
    n1, n2m = b1["node_mlp"], b2["node_mlp"]
    proj = params["proj"]

    x0, ts1, td1 = _tc_node_encode(
        cin8, we1, _row(enc[0]["b"]), enc[1]["W"], _row(enc[1]["b"]),
        enc[2]["W"], _row(enc[2]["b"]),
        e1[0]["W"][0:WID], e1[0]["W"][WID:2 * WID])

    gs1, gd1, relx, rely, relz, d2 = _sc_gather1(
        ts1, td1, c2[:, 0], c2[:, 1], c2[:, 2], isrc, idst)

    u1 = _tc_edge1(
        gs1, gd1, relx.reshape(E2, 1), rely.reshape(E2, 1),
        relz.reshape(E2, 1), d2.reshape(E2, 1),
        wr0, wr1, wr2, wd, _row(eenc[0]["b"]),
        eenc[1]["W"], _row(eenc[1]["b"]), eenc[2]["W"], _row(eenc[2]["b"]),
        e1[0]["W"][2 * WID:], _row(e1[0]["b"]),
        e1[1]["W"], _row(e1[1]["b"]), e1[2]["W"], _row(e1[2]["b"]),
        _row(b1["edge_norm"]["g"]), _row(b1["edge_norm"]["b"]))

    s1 = _sc_scatter_add(u1, idst)
    dg = _sc_degree(idst)

    x1, p2, q2 = _tc_node1(
        x0, s1, dg,
        n1[0]["W"][0:WID], n1[0]["W"][WID:], _row(n1[0]["b"]),
        n1[1]["W"], _row(n1[1]["b"]), n1[2]["W"], _row(n1[2]["b"]),
        _row(b1["node_norm"]["g"]), _row(b1["node_norm"]["b"]),
        e2m[0]["W"][0:WID], e2m[0]["W"][WID:2 * WID])

    gs2, gd2 = _sc_gather_pair(p2, q2, isrc, idst)

    u2 = _tc_edge2(
        u1, gs2, gd2,
        e2m[0]["W"][2 * WID:], _row(e2m[0]["b"]),
        e2m[1]["W"], _row(e2m[1]["b"]), e2m[2]["W"], _row(e2m[2]["b"]),
        _row(b2["edge_norm"]["g"]), _row(b2["edge_norm"]["b"]))

    s2 = _sc_scatter_add(u2, idst)

    out = _tc_node2(
        x1, s2, dg,
        n2m[0]["W"][0:WID], n2m[0]["W"][WID:], _row(n2m[0]["b"]),
        n2m[1]["W"], _row(n2m[1]["b"]), n2m[2]["W"], _row(n2m[2]["b"]),
        _row(b2["node_norm"]["g"]), _row(b2["node_norm"]["b"]),
        proj[0]["W"], _row(proj[0]["b"]), proj[1]["W"], _row(proj[1]["b"]))

    return out.reshape(1, N, 4)


# speedup vs baseline: 5.5490x; 1.0997x over previous
import dataclasses
import functools

import jax
import jax.numpy as jnp
from jax.experimental import pallas as pl
from jax.experimental.pallas import tpu as pltpu
from jax.experimental.pallas import tpu_sc as plsc

N = 10000
E = 160000
E2 = 2 * E
WID = 128
GW1 = 128
GW2 = 128
SW = 128
ER = 2000

_F32 = jnp.float32


def _vmesh():
    return plsc.VectorSubcoreMesh(core_axis_name="c", subcore_axis_name="s")


def _sc_params():
    cp = pltpu.CompilerParams()
    if "needs_layout_passes" in pltpu.CompilerParams.__dataclass_fields__:
        cp = dataclasses.replace(cp, needs_layout_passes=False)
    return cp


def _sc_gather_pair(tsrc, tdst, isrc, idst):
    tw = tsrc.shape[1]
    n_idx = isrc.shape[1]
    tdt = tsrc.dtype

    @pl.kernel(
        out_type=[
            jax.ShapeDtypeStruct((n_idx, tw), tdt),
            jax.ShapeDtypeStruct((n_idx, tw), tdt),
        ],
        mesh=_vmesh(),
    )
    def gk(tsrc_hbm, tdst_hbm, isrc_hbm, idst_hbm, osrc_hbm, odst_hbm):
        def body(isrc_v, idst_v, osrc_v, odst_v):
            pltpu.sync_copy(tsrc_hbm.at[isrc_v.at[0]], osrc_v)
            pltpu.sync_copy(tdst_hbm.at[idst_v.at[0]], odst_v)

        pltpu.emit_pipeline(
            body,
            grid=(n_idx // GW2,),
            in_specs=[
                pl.BlockSpec((1, GW2), lambda i: (0, i)),
                pl.BlockSpec((1, GW2), lambda i: (0, i)),
            ],
            out_specs=[
                pl.BlockSpec((GW2, tw), lambda i: (i, 0)),
                pl.BlockSpec((GW2, tw), lambda i: (i, 0)),
            ],
            core_axis_name=("c", "s"),
            dimension_semantics=(pltpu.PARALLEL,),
        )(isrc_hbm, idst_hbm, osrc_hbm, odst_hbm)

    return gk(tsrc, tdst, isrc, idst)


def _sc_gather1(tsrc, tdst, cx, cy, cz, isrc, idst):
    tw = tsrc.shape[1]
    n_idx = isrc.shape[1]
    tdt = tsrc.dtype
    geo = jax.ShapeDtypeStruct((n_idx,), _F32)

    @pl.kernel(
        out_type=[
            jax.ShapeDtypeStruct((n_idx, tw), tdt),
            jax.ShapeDtypeStruct((n_idx, tw), tdt),
            geo, geo, geo, geo,
        ],
        mesh=_vmesh(),
        scratch_types=[
            pltpu.VMEM((N,), _F32),
            pltpu.VMEM((N,), _F32),
            pltpu.VMEM((N,), _F32),
        ],
        compiler_params=_sc_params(),
    )
    def gk(tsrc_hbm, tdst_hbm, cx_hbm, cy_hbm, cz_hbm, isrc_hbm, idst_hbm,
           osrc_hbm, odst_hbm, orx_hbm, ory_hbm, orz_hbm, od2_hbm,
           cx_v, cy_v, cz_v):
        pltpu.sync_copy(cx_hbm, cx_v)
        pltpu.sync_copy(cy_hbm, cy_v)
        pltpu.sync_copy(cz_hbm, cz_v)

        def body(isrc_v, idst_v, osrc_v, odst_v, orx_v, ory_v, orz_v, od2_v):
            pltpu.sync_copy(tsrc_hbm.at[isrc_v.at[0]], osrc_v)
            pltpu.sync_copy(tdst_hbm.at[idst_v.at[0]], odst_v)

            @pl.loop(0, GW1, step=16)
            def _(c):
                s16 = isrc_v[0, pl.ds(c, 16)]
                d16 = idst_v[0, pl.ds(c, 16)]
                rx = (plsc.load_gather(cx_v, [d16])
                      - plsc.load_gather(cx_v, [s16]))
                ry = (plsc.load_gather(cy_v, [d16])
                      - plsc.load_gather(cy_v, [s16]))
                rz = (plsc.load_gather(cz_v, [d16])
                      - plsc.load_gather(cz_v, [s16]))
                orx_v[pl.ds(c, 16)] = rx
                ory_v[pl.ds(c, 16)] = ry
                orz_v[pl.ds(c, 16)] = rz
                od2_v[pl.ds(c, 16)] = rx * rx + ry * ry + rz * rz

        pltpu.emit_pipeline(
            body,
            grid=(n_idx // GW1,),
            in_specs=[
                pl.BlockSpec((1, GW1), lambda i: (0, i)),
                pl.BlockSpec((1, GW1), lambda i: (0, i)),
            ],
            out_specs=[
                pl.BlockSpec((GW1, tw), lambda i: (i, 0)),
                pl.BlockSpec((GW1, tw), lambda i: (i, 0)),
                pl.BlockSpec((GW1,), lambda i: (i,)),
                pl.BlockSpec((GW1,), lambda i: (i,)),
                pl.BlockSpec((GW1,), lambda i: (i,)),
                pl.BlockSpec((GW1,), lambda i: (i,)),
            ],
            core_axis_name=("c", "s"),
            dimension_semantics=(pltpu.PARALLEL,),
        )(isrc_hbm, idst_hbm, osrc_hbm, odst_hbm,
          orx_hbm, ory_hbm, orz_hbm, od2_hbm)

    return gk(tsrc, tdst, cx, cy, cz, isrc, idst)


def _sc_scatter_add(u, idx):
    n_idx = idx.shape[1]
    z_tab = jnp.zeros((N, WID), _F32)

    @pl.kernel(
        out_type=jax.ShapeDtypeStruct((2, N, WID), _F32),
        mesh=_vmesh(),
        scratch_types=[pltpu.VMEM_SHARED((N, WID), _F32)],
    )
    def sk(u_hbm, idx_hbm, z_hbm, s_hbm, tab):
        core = jax.lax.axis_index("c")
        sid = jax.lax.axis_index("s")

        @pl.when(sid == 0)
        def _():
            pltpu.sync_copy(z_hbm, tab)

        plsc.subcore_barrier()

        def body(u_v, idx_v):
            pltpu.sync_copy(u_v, tab.at[idx_v.at[0]], add=True)

        pltpu.emit_pipeline(
            body,
            grid=(n_idx // SW,),
            in_specs=[
                pl.BlockSpec((SW, WID), lambda i: (i, 0)),
                pl.BlockSpec((1, SW), lambda i: (0, i)),
            ],
            core_axis_name=("c", "s"),
            dimension_semantics=(pltpu.PARALLEL,),
        )(u_hbm, idx_hbm)

        plsc.subcore_barrier()

        @pl.when(sid == 0)
        def _():
            pltpu.sync_copy(tab, s_hbm.at[core])

    return sk(u, idx, z_tab)


def _sc_degree(idx):
    n_idx = idx.shape[1]
    z_deg = jnp.zeros((N, WID), _F32)
    ones = jnp.ones((SW, WID), _F32)

    @pl.kernel(
        out_type=jax.ShapeDtypeStruct((2, N, WID), _F32),
        mesh=_vmesh(),
        scratch_types=[pltpu.VMEM_SHARED((N, WID), _F32)],
    )
    def dk(ones_hbm, idx_hbm, zd_hbm, d_hbm, dtab):
        core = jax.lax.axis_index("c")
        sid = jax.lax.axis_index("s")

        @pl.when(sid == 0)
        def _():
            pltpu.sync_copy(zd_hbm, dtab)

        plsc.subcore_barrier()

        def body(ones_v, idx_v):
            pltpu.sync_copy(ones_v, dtab.at[idx_v.at[0]], add=True)

        pltpu.emit_pipeline(
            body,
            grid=(n_idx // SW,),
            in_specs=[
                pl.BlockSpec((SW, WID), lambda i: (0, 0)),
                pl.BlockSpec((1, SW), lambda i: (0, i)),
            ],
            core_axis_name=("c", "s"),
            dimension_semantics=(pltpu.PARALLEL,),
        )(ones_hbm, idx_hbm)

        plsc.subcore_barrier()

        @pl.when(sid == 0)
        def _():
            pltpu.sync_copy(dtab, d_hbm.at[core])

    return dk(ones, idx, z_deg)


def _gelu(x):
    return x * 0.5 * (1.0 + jax.lax.erf(x * 0.7071067811865476))


def _dot(a, b):
    return jax.lax.dot_general(a, b, (((1,), (0,)), ((), ())),
                               preferred_element_type=_F32)


def _ln(u, g, b):
    mu = jnp.mean(u, axis=-1, keepdims=True)
    var = jnp.mean((u - mu) * (u - mu), axis=-1, keepdims=True)
    return (u - mu) * jax.lax.rsqrt(var + 1e-5) * g + b


def _full_spec(shape):
    return pl.BlockSpec(shape, lambda *_: tuple(0 for _ in shape))


def _tc_node_encode(cin8, we1, be1, we2, be2, we3, be3, w1a, w1b):
    def body(cin_r, we1_r, be1_r, we2_r, be2_r, we3_r, be3_r,
             w1a_r, w1b_r, x0_r, ts_r, td_r):
        h = _gelu(_dot(cin_r[...], we1_r[...]) + be1_r[...])
        h = _gelu(_dot(h, we2_r[...]) + be2_r[...])
        x0 = _dot(h, we3_r[...]) + be3_r[...]
        x0_r[...] = x0
        ts_r[...] = _dot(x0, w1a_r[...])
        td_r[...] = _dot(x0, w1b_r[...])

    return pl.pallas_call(
        body,
        out_shape=[
            jax.ShapeDtypeStruct((N, WID), _F32),
            jax.ShapeDtypeStruct((N, WID), _F32),
            jax.ShapeDtypeStruct((N, WID), _F32),
        ],
    )(cin8, we1, be1, we2, be2, we3, be3, w1a, w1b)


def _tc_edge1(gsrc, gdst, relx, rely, relz, d2,
              wr0, wr1, wr2, wd, bee1, we2, be2, we3, be3,
              w1c, b1, w2, b2, w3, b3, g, bn):
    def body(gs_r, gd_r, rx_r, ry_r, rz_r, d2_r,
             wr0_r, wr1_r, wr2_r, wd_r, bee1_r, we2_r, be2_r, we3_r, be3_r,
             w1c_r, b1_r, w2_r, b2_r, w3_r, b3_r, g_r, bn_r, u_r):
        dist = jnp.sqrt(d2_r[...])
        h = _gelu(rx_r[...] * wr0_r[...] + ry_r[...] * wr1_r[...]
                  + rz_r[...] * wr2_r[...] + dist * wd_r[...] + bee1_r[...])
        h = _gelu(_dot(h, we2_r[...]) + be2_r[...])
        ea = _dot(h, we3_r[...]) + be3_r[...]
        z = _gelu(_dot(ea, w1c_r[...]) + gs_r[...] + gd_r[...] + b1_r[...])
        z = _gelu(_dot(z, w2_r[...]) + b2_r[...])
        u = _dot(z, w3_r[...]) + b3_r[...]
        u_r[...] = _ln(u, g_r[...], bn_r[...])

    ne = gsrc.shape[0]
    col = pl.BlockSpec((ER, 1), lambda i: (i, 0))
    return pl.pallas_call(
        body,
        grid=(ne // ER,),
        in_specs=[
            pl.BlockSpec((ER, WID), lambda i: (i, 0)),
            pl.BlockSpec((ER, WID), lambda i: (i, 0)),
            col, col, col, col,
            _full_spec((1, WID)), _full_spec((1, WID)), _full_spec((1, WID)),
            _full_spec((1, WID)), _full_spec((1, WID)),
            _full_spec((WID, WID)), _full_spec((1, WID)),
            _full_spec((WID, WID)), _full_spec((1, WID)),
            _full_spec((WID, WID)), _full_spec((1, WID)),
            _full_spec((WID, WID)), _full_spec((1, WID)),
            _full_spec((WID, WID)), _full_spec((1, WID)),
            _full_spec((1, WID)), _full_spec((1, WID)),
        ],
        out_specs=pl.BlockSpec((ER, WID), lambda i: (i, 0)),
        out_shape=jax.ShapeDtypeStruct((ne, WID), _F32),
    )(gsrc, gdst, relx, rely, relz, d2,
      wr0, wr1, wr2, wd, bee1, we2, be2, we3, be3,
      w1c, b1, w2, b2, w3, b3, g, bn)


def _tc_edge2(attr, gsrc, gdst, w1c, b1, w2, b2, w3, b3, g, bn):
    def body(a_r, gs_r, gd_r, w1c_r, b1_r, w2_r, b2_r, w3_r, b3_r,
             g_r, bn_r, u_r):
        z = _gelu(_dot(a_r[...], w1c_r[...]) + gs_r[...] + gd_r[...]
                  + b1_r[...])
        z = _gelu(_dot(z, w2_r[...]) + b2_r[...])
        u = _dot(z, w3_r[...]) + b3_r[...]
        u_r[...] = _ln(u, g_r[...], bn_r[...])

    ne = attr.shape[0]
    return pl.pallas_call(
        body,
        grid=(ne // ER,),
        in_specs=[
            pl.BlockSpec((ER, WID), lambda i: (i, 0)),
            pl.BlockSpec((ER, WID), lambda i: (i, 0)),
            pl.BlockSpec((ER, WID), lambda i: (i, 0)),
            _full_spec((WID, WID)), _full_spec((1, WID)),
            _full_spec((WID, WID)), _full_spec((1, WID)),
            _full_spec((WID, WID)), _full_spec((1, WID)),
            _full_spec((1, WID)), _full_spec((1, WID)),
        ],
        out_specs=pl.BlockSpec((ER, WID), lambda i: (i, 0)),
        out_shape=jax.ShapeDtypeStruct((ne, WID), _F32),
    )(attr, gsrc, gdst, w1c, b1, w2, b2, w3, b3, g, bn)


def _tc_node1(x0, sa, sb, dg, wna, wnb, bn1, wn2, bn2, wn3, bn3, g, b,
              w1a2, w1b2):
    def body(x_r, sa_r, sb_r, dg_r, wna_r, wnb_r, bn1_r, wn2_r, bn2_r, wn3_r,
             bn3_r, g_r, b_r, w1a_r, w1b_r, x1_r, p_r, q_r):
        x0v = x_r[...]
        deg = dg_r[0, :, 0:1] + dg_r[1, :, 0:1]
        agg = ((sa_r[0] + sa_r[1] + sb_r[0] + sb_r[1])
               / jnp.maximum(deg, 1.0))
        h = _gelu(_dot(x0v, wna_r[...]) + _dot(agg, wnb_r[...]) + bn1_r[...])
        h = _gelu(_dot(h, wn2_r[...]) + bn2_r[...])
        nu = _dot(h, wn3_r[...]) + bn3_r[...]
        x1 = x0v + _ln(nu, g_r[...], b_r[...])
        x1_r[...] = x1
        p_r[...] = _dot(x1, w1a_r[...])
        q_r[...] = _dot(x1, w1b_r[...])

    return pl.pallas_call(
        body,
        out_shape=[
            jax.ShapeDtypeStruct((N, WID), _F32),
            jax.ShapeDtypeStruct((N, WID), _F32),
            jax.ShapeDtypeStruct((N, WID), _F32),
        ],
    )(x0, sa, sb, dg, wna, wnb, bn1, wn2, bn2, wn3, bn3, g, b, w1a2, w1b2)


def _tc_node2(x1, sa, sb, dg, wna, wnb, bn1, wn2, bn2, wn3, bn3, g, b,
              wp1, bp1, wp2, bp2):
    def body(x_r, sa_r, sb_r, dg_r, wna_r, wnb_r, bn1_r, wn2_r, bn2_r, wn3_r,
             bn3_r, g_r, b_r, wp1_r, bp1_r, wp2_r, bp2_r, o_r):
        x1v = x_r[...]
        deg = dg_r[0, :, 0:1] + dg_r[1, :, 0:1]
        agg = ((sa_r[0] + sa_r[1] + sb_r[0] + sb_r[1])
               / jnp.maximum(deg, 1.0))
        h = _gelu(_dot(x1v, wna_r[...]) + _dot(agg, wnb_r[...]) + bn1_r[...])
        h = _gelu(_dot(h, wn2_r[...]) + bn2_r[...])
        nu = _dot(h, wn3_r[...]) + bn3_r[...]
        x2 = x1v + _ln(nu, g_r[...], b_r[...])
        o = _gelu(_dot(x2, wp1_r[...]) + bp1_r[...])
        o_r[...] = _dot(o, wp2_r[...]) + bp2_r[...]

    return pl.pallas_call(
        body,
        out_shape=jax.ShapeDtypeStruct((N, 4), _F32),
    )(x1, sa, sb, dg, wna, wnb, bn1, wn2, bn2, wn3, bn3, g, b,
      wp1, bp1, wp2, bp2)


def _row(v):
    return v.reshape(1, -1)


def kernel(inputs, coords, edge_index, params):
    c2 = coords[0]
    f2 = inputs[0]

    ei = edge_index.astype(jnp.int32)
    src = jnp.concatenate([ei[0], ei[1]])
    dst = jnp.concatenate([ei[1], ei[0]])
    isrc = src.reshape(1, E2)
    idst = dst.reshape(1, E2)

    cin8 = jnp.concatenate([c2, f2, jnp.zeros((N, 1), _F32)], axis=-1)

    enc = params["node_encoder"]
    we1 = jnp.concatenate([enc[0]["W"], jnp.zeros((1, WID), _F32)], axis=0)
    eenc = params["edge_encoder"]
    wr0 = _row(eenc[0]["W"][0])
    wr1 = _row(eenc[0]["W"][1])
    wr2 = _row(eenc[0]["W"][2])
    wd = _row(eenc[0]["W"][3])

    b1, b2 = params["blocks"]
    e1, e2m = b1["edge_mlp"], b2["edge_mlp"]
    n1, n2m = b1["node_mlp"], b2["node_mlp"]
    proj = params["proj"]

    x0, ts1, td1 = _tc_node_encode(
        cin8, we1, _row(enc[0]["b"]), enc[1]["W"], _row(enc[1]["b"]),
        enc[2]["W"], _row(enc[2]["b"]),
        e1[0]["W"][0:WID], e1[0]["W"][WID:2 * WID])

    half = E2 // 2
    is_c = [isrc[:, :half], isrc[:, half:]]
    id_c = [idst[:, :half], idst[:, half:]]

    dg = _sc_degree(idst)

    u1 = []
    s1 = []
    for c in range(2):
        gs1, gd1, relx, rely, relz, d2 = _sc_gather1(
            ts1, td1, c2[:, 0], c2[:, 1], c2[:, 2], is_c[c], id_c[c])
        u1.append(_tc_edge1(
            gs1, gd1, relx.reshape(half, 1), rely.reshape(half, 1),
            relz.reshape(half, 1), d2.reshape(half, 1),
            wr0, wr1, wr2, wd, _row(eenc[0]["b"]),
            eenc[1]["W"], _row(eenc[1]["b"]), eenc[2]["W"],
            _row(eenc[2]["b"]),
            e1[0]["W"][2 * WID:], _row(e1[0]["b"]),
            e1[1]["W"], _row(e1[1]["b"]), e1[2]["W"], _row(e1[2]["b"]),
            _row(b1["edge_norm"]["g"]), _row(b1["edge_norm"]["b"])))
        s1.append(_sc_scatter_add(u1[c], id_c[c]))

    x1, p2, q2 = _tc_node1(
        x0, s1[0], s1[1], dg,
        n1[0]["W"][0:WID], n1[0]["W"][WID:], _row(n1[0]["b"]),
        n1[1]["W"], _row(n1[1]["b"]), n1[2]["W"], _row(n1[2]["b"]),
        _row(b1["node_norm"]["g"]), _row(b1["node_norm"]["b"]),
        e2m[0]["W"][0:WID], e2m[0]["W"][WID:2 * WID])

    s2 = []
    for c in range(2):
        gs2, gd2 = _sc_gather_pair(p2, q2, is_c[c], id_c[c])
        u2 = _tc_edge2(
            u1[c], gs2, gd2,
            e2m[0]["W"][2 * WID:], _row(e2m[0]["b"]),
            e2m[1]["W"], _row(e2m[1]["b"]), e2m[2]["W"], _row(e2m[2]["b"]),
            _row(b2["edge_norm"]["g"]), _row(b2["edge_norm"]["b"]))
        s2.append(_sc_scatter_add(u2, id_c[c]))

    out = _tc_node2(
        x1, s2[0], s2[1], dg,
        n2m[0]["W"][0:WID], n2m[0]["W"][WID:], _row(n2m[0]["b"]),
        n2m[1]["W"], _row(n2m[1]["b"]), n2m[2]["W"], _row(n2m[2]["b"]),
        _row(b2["node_norm"]["g"]), _row(b2["node_norm"]["b"]),
        proj[0]["W"], _row(proj[0]["b"]), proj[1]["W"], _row(proj[1]["b"]))

    return out.reshape(1, N, 4)
